# Initial kernel scaffold; baseline (speedup 1.0000x reference)
import jax
import jax.numpy as jnp
from jax.experimental import pallas as pl


def kernel(x, edge_index, W1, b1, W2, b2, Wfc, bfc):
    raise NotImplementedError("write your pallas kernel here")


# speedup vs baseline: 23.4011x; 23.4011x over previous
import functools

import jax
import jax.numpy as jnp
from jax import lax
from jax.experimental import pallas as pl
from jax.experimental.pallas import tpu as pltpu
from jax.experimental.pallas import tpu_sc as plsc

N = 10000
E = 320000
D = 128
NC = 2
NS = 16
NW = NC * NS
CH = 125
ROWS_W = E // (NW * CH)
NP = 10240
STRIPE = NP // NS

_mesh = plsc.VectorSubcoreMesh(
    core_axis_name="c", subcore_axis_name="s", num_cores=NC, num_subcores=NS
)


@functools.partial(
    pl.kernel,
    out_type=jax.ShapeDtypeStruct((NC, N), jnp.float32),
    mesh=_mesh,
    scratch_types=[
        pltpu.VMEM((ROWS_W, CH), jnp.int32),
        pltpu.VMEM((128,), jnp.float32),
        pltpu.VMEM((2048,), jnp.float32),
        pltpu.VMEM_SHARED((N,), jnp.float32),
    ],
)
def _sc_counts(dst_hbm, out_hbm, dst_v, ones_v, z_v, cnt_sp):
    cid = lax.axis_index("c")
    sid = lax.axis_index("s")
    wid = sid * NC + cid

    for i in range(8):
        ones_v[pl.ds(i * 16, 16)] = jnp.ones((16,), jnp.float32)

    @pl.when(sid == 0)
    def _zero():
        def zfill(r, carry):
            z_v[pl.ds(r * 16, 16)] = jnp.zeros((16,), jnp.float32)
            return carry
        lax.fori_loop(0, 128, zfill, 0)

        def zcopy(k, carry):
            pltpu.sync_copy(z_v.at[pl.ds(0, 2000)],
                            cnt_sp.at[pl.ds(k * 2000, 2000)])
            return carry
        lax.fori_loop(0, 5, zcopy, 0)

    pltpu.sync_copy(dst_hbm.at[pl.ds(wid * ROWS_W, ROWS_W), :], dst_v)
    plsc.subcore_barrier()

    def body(j, carry):
        pltpu.sync_copy(ones_v.at[pl.ds(0, CH)], cnt_sp.at[dst_v.at[j]],
                        add=True)
        return carry
    lax.fori_loop(0, ROWS_W, body, 0)

    plsc.subcore_barrier()

    @pl.when(sid == 0)
    def _out():
        pltpu.sync_copy(cnt_sp, out_hbm.at[cid])


@functools.partial(
    pl.kernel,
    out_type=jax.ShapeDtypeStruct((NC, NP, D), jnp.float32),
    mesh=_mesh,
    scratch_types=[
        pltpu.VMEM((ROWS_W, CH), jnp.int32),
        pltpu.VMEM((ROWS_W, CH), jnp.int32),
        pltpu.VMEM((128, D), jnp.float32),
        pltpu.VMEM_SHARED((NP, D), jnp.float32),
        pltpu.SemaphoreType.DMA,
    ],
)
def _sc_scatter(g_hbm, src2_hbm, dst2_hbm, out_hbm,
                src_v, dst_v, rows_v, acc_sp, sem):
    cid = lax.axis_index("c")
    sid = lax.axis_index("s")
    wid = sid * NC + cid

    def zfill(r, carry):
        for c in range(D // 16):
            rows_v[r, pl.ds(c * 16, 16)] = jnp.zeros((16,), jnp.float32)
        return carry
    lax.fori_loop(0, 128, zfill, 0)

    def zcopy(k, carry):
        pltpu.sync_copy(rows_v,
                        acc_sp.at[pl.ds(sid * STRIPE + k * 128, 128), :])
        return carry
    lax.fori_loop(0, 5, zcopy, 0)

    pltpu.sync_copy(src2_hbm.at[pl.ds(wid * ROWS_W, ROWS_W), :], src_v)
    pltpu.sync_copy(dst2_hbm.at[pl.ds(wid * ROWS_W, ROWS_W), :], dst_v)
    plsc.subcore_barrier()

    def body(j, carry):
        pltpu.async_copy(g_hbm.at[src_v.at[j]],
                         rows_v.at[pl.ds(0, CH), :], sem).wait()
        pltpu.sync_copy(rows_v.at[pl.ds(0, CH), :],
                        acc_sp.at[dst_v.at[j]], add=True)
        return carry
    lax.fori_loop(0, ROWS_W, body, 0)

    plsc.subcore_barrier()
    pltpu.sync_copy(acc_sp.at[pl.ds(sid * STRIPE, STRIPE), :],
                    out_hbm.at[cid, pl.ds(sid * STRIPE, STRIPE), :])


BR = 1000


def _dinv_block(cnt_ref):
    c = cnt_ref[...]
    deg = c[:, 0] + c[:, 1] + 1.0
    return lax.rsqrt(deg)[:, None]


def _tc_scale_matmul_body(cnt_ref, x_ref, w_ref, g_ref):
    h = jnp.dot(x_ref[...], w_ref[...], preferred_element_type=jnp.float32)
    g_ref[...] = _dinv_block(cnt_ref) * h


def _tc_mid_body(cnt_ref, acc_ref, g_ref, b_ref, w_ref, g2_ref):
    dinv = _dinv_block(cnt_ref)
    a = acc_ref[0] + acc_ref[1] + g_ref[...]
    z = jnp.maximum(dinv * a + b_ref[...], 0.0)
    g2_ref[...] = dinv * jnp.dot(z, w_ref[...],
                                 preferred_element_type=jnp.float32)


def _tc_final_body(cnt_ref, acc_ref, g_ref, b_ref, wfc_ref, bfc_ref, o_ref):
    dinv = _dinv_block(cnt_ref)
    a = acc_ref[0] + acc_ref[1] + g_ref[...]
    z = jnp.maximum(dinv * a + b_ref[...], 0.0)
    o_ref[...] = jnp.dot(z, wfc_ref[...],
                         preferred_element_type=jnp.float32) + bfc_ref[...]


_cnt_spec = pl.BlockSpec((BR, 2), lambda i: (i, 0))
_row_spec = pl.BlockSpec((BR, D), lambda i: (i, 0))
_acc_spec = pl.BlockSpec((2, BR, D), lambda i: (0, i, 0))
_w_spec = pl.BlockSpec((D, D), lambda i: (0, 0))
_b_spec = pl.BlockSpec((1, D), lambda i: (0, 0))

_tc_scale_matmul = pl.pallas_call(
    _tc_scale_matmul_body,
    grid=(N // BR,),
    in_specs=[_cnt_spec, _row_spec, _w_spec],
    out_specs=_row_spec,
    out_shape=jax.ShapeDtypeStruct((N, D), jnp.float32),
)

_tc_mid = pl.pallas_call(
    _tc_mid_body,
    grid=(N // BR,),
    in_specs=[_cnt_spec, _acc_spec, _row_spec, _b_spec, _w_spec],
    out_specs=_row_spec,
    out_shape=jax.ShapeDtypeStruct((N, D), jnp.float32),
)

_tc_final = pl.pallas_call(
    _tc_final_body,
    grid=(N // BR,),
    in_specs=[_cnt_spec, _acc_spec, _row_spec, _b_spec,
              pl.BlockSpec((D, 8), lambda i: (0, 0)),
              pl.BlockSpec((1, 8), lambda i: (0, 0))],
    out_specs=pl.BlockSpec((BR, 8), lambda i: (i, 0)),
    out_shape=jax.ShapeDtypeStruct((N, 8), jnp.float32),
)


def kernel(x, edge_index, W1, b1, W2, b2, Wfc, bfc):
    src2 = edge_index[0].reshape(E // CH, CH)
    dst2 = edge_index[1].reshape(E // CH, CH)
    wfc_p = jnp.zeros((D, 8), jnp.float32).at[:, :4].set(Wfc)
    bfc_p = jnp.zeros((1, 8), jnp.float32).at[0, :4].set(bfc)

    cnt = _sc_counts(dst2).T

    g1 = _tc_scale_matmul(cnt, x, W1)
    acc1 = _sc_scatter(g1, src2, dst2)
    g2 = _tc_mid(cnt, acc1, g1, b1.reshape(1, D), W2)
    acc2 = _sc_scatter(g2, src2, dst2)
    out = _tc_final(cnt, acc2, g2, b2.reshape(1, D), wfc_p, bfc_p)
    return out[:, :4]


# speedup vs baseline: 33.6749x; 1.4390x over previous
import functools

import jax
import jax.numpy as jnp
from jax import lax
from jax.experimental import pallas as pl
from jax.experimental.pallas import tpu as pltpu
from jax.experimental.pallas import tpu_sc as plsc

N = 10000
E = 320000
D = 128
NC = 2
NS = 16
NW = NC * NS
CH = 125
ROWS_W = E // (NW * CH)
NH = 2
HROWS = ROWS_W // NH
NPAIR = HROWS // 2
NP = 10240
STRIPE = NP // NS

_mesh = plsc.VectorSubcoreMesh(
    core_axis_name="c", subcore_axis_name="s", num_cores=NC, num_subcores=NS
)


@functools.partial(
    pl.kernel,
    out_type=jax.ShapeDtypeStruct((NC, N), jnp.float32),
    mesh=_mesh,
    scratch_types=[
        pltpu.VMEM((ROWS_W, CH), jnp.int32),
        pltpu.VMEM((128,), jnp.float32),
        pltpu.VMEM((2048,), jnp.float32),
        pltpu.VMEM_SHARED((N,), jnp.float32),
    ],
)
def _sc_counts(dst_hbm, out_hbm, dst_v, ones_v, z_v, cnt_sp):
    cid = lax.axis_index("c")
    sid = lax.axis_index("s")
    wid = sid * NC + cid

    for i in range(8):
        ones_v[pl.ds(i * 16, 16)] = jnp.ones((16,), jnp.float32)

    @pl.when(sid == 0)
    def _zero():
        def zfill(r, carry):
            z_v[pl.ds(r * 16, 16)] = jnp.zeros((16,), jnp.float32)
            return carry
        lax.fori_loop(0, 128, zfill, 0)

        def zcopy(k, carry):
            pltpu.sync_copy(z_v.at[pl.ds(0, 2000)],
                            cnt_sp.at[pl.ds(k * 2000, 2000)])
            return carry
        lax.fori_loop(0, 5, zcopy, 0)

    pltpu.sync_copy(dst_hbm.at[wid], dst_v)
    plsc.subcore_barrier()

    def body(j, carry):
        pltpu.sync_copy(ones_v.at[pl.ds(0, CH)], cnt_sp.at[dst_v.at[j]],
                        add=True)
        return carry
    lax.fori_loop(0, ROWS_W, body, 0)

    plsc.subcore_barrier()

    @pl.when(sid == 0)
    def _out():
        pltpu.sync_copy(cnt_sp, out_hbm.at[cid])


@functools.partial(
    pl.kernel,
    out_type=jax.ShapeDtypeStruct((NC, NP, D), jnp.float32),
    mesh=_mesh,
    scratch_types=[
        pltpu.VMEM((HROWS, CH), jnp.int32),
        pltpu.VMEM((HROWS, CH), jnp.int32),
        pltpu.VMEM((128, D), jnp.float32),
        pltpu.VMEM((128, D), jnp.float32),
        pltpu.VMEM_SHARED((NP, D), jnp.float32),
        pltpu.SemaphoreType.DMA,
        pltpu.SemaphoreType.DMA,
    ],
)
def _sc_scatter(g_hbm, src3_hbm, dst3_hbm, out_hbm,
                src_v, dst_v, buf0, buf1, acc_sp, sem0, sem1):
    cid = lax.axis_index("c")
    sid = lax.axis_index("s")
    wid = sid * NC + cid
    b0 = buf0.at[pl.ds(0, CH), :]
    b1 = buf1.at[pl.ds(0, CH), :]

    def zfill(r, carry):
        for c in range(D // 16):
            buf0[r, pl.ds(c * 16, 16)] = jnp.zeros((16,), jnp.float32)
        return carry
    lax.fori_loop(0, 128, zfill, 0)

    def zcopy(k, carry):
        pltpu.sync_copy(buf0,
                        acc_sp.at[pl.ds(sid * STRIPE + k * 128, 128), :])
        return carry
    lax.fori_loop(0, 5, zcopy, 0)
    plsc.subcore_barrier()

    for h in range(NH):
        pltpu.sync_copy(src3_hbm.at[wid, pl.ds(h * HROWS, HROWS), :], src_v)
        pltpu.sync_copy(dst3_hbm.at[wid, pl.ds(h * HROWS, HROWS), :], dst_v)
        pltpu.async_copy(g_hbm.at[src_v.at[0]], b0, sem0)

        def body(p, carry):
            j = 2 * p
            pltpu.async_copy(g_hbm.at[src_v.at[j + 1]], b1, sem1)
            pltpu.make_async_copy(g_hbm.at[src_v.at[j]], b0, sem0).wait()
            pltpu.sync_copy(b0, acc_sp.at[dst_v.at[j]], add=True)

            @pl.when(p < NPAIR - 1)
            def _():
                pltpu.async_copy(g_hbm.at[src_v.at[j + 2]], b0, sem0)

            pltpu.make_async_copy(g_hbm.at[src_v.at[j + 1]], b1, sem1).wait()
            pltpu.sync_copy(b1, acc_sp.at[dst_v.at[j + 1]], add=True)
            return carry
        lax.fori_loop(0, NPAIR, body, 0)

    plsc.subcore_barrier()
    pltpu.sync_copy(acc_sp.at[pl.ds(sid * STRIPE, STRIPE), :],
                    out_hbm.at[cid, pl.ds(sid * STRIPE, STRIPE), :])


BR = 1000


def _dinv_block(cnt_ref):
    c = cnt_ref[...]
    deg = c[:, 0] + c[:, 1] + 1.0
    return lax.rsqrt(deg)[:, None]


def _tc_scale_matmul_body(cnt_ref, x_ref, w_ref, g_ref):
    h = jnp.dot(x_ref[...], w_ref[...], preferred_element_type=jnp.float32)
    g_ref[...] = _dinv_block(cnt_ref) * h


def _tc_mid_body(cnt_ref, acc_ref, g_ref, b_ref, w_ref, g2_ref):
    dinv = _dinv_block(cnt_ref)
    a = acc_ref[0] + acc_ref[1] + g_ref[...]
    z = jnp.maximum(dinv * a + b_ref[...], 0.0)
    g2_ref[...] = dinv * jnp.dot(z, w_ref[...],
                                 preferred_element_type=jnp.float32)


def _tc_final_body(cnt_ref, acc_ref, g_ref, b_ref, wfc_ref, bfc_ref, o_ref):
    dinv = _dinv_block(cnt_ref)
    a = acc_ref[0] + acc_ref[1] + g_ref[...]
    z = jnp.maximum(dinv * a + b_ref[...], 0.0)
    o_ref[...] = jnp.dot(z, wfc_ref[...],
                         preferred_element_type=jnp.float32) + bfc_ref[...]


_cnt_spec = pl.BlockSpec((BR, 2), lambda i: (i, 0))
_row_spec = pl.BlockSpec((BR, D), lambda i: (i, 0))
_acc_spec = pl.BlockSpec((2, BR, D), lambda i: (0, i, 0))
_w_spec = pl.BlockSpec((D, D), lambda i: (0, 0))
_b_spec = pl.BlockSpec((1, D), lambda i: (0, 0))

_tc_scale_matmul = pl.pallas_call(
    _tc_scale_matmul_body,
    grid=(N // BR,),
    in_specs=[_cnt_spec, _row_spec, _w_spec],
    out_specs=_row_spec,
    out_shape=jax.ShapeDtypeStruct((N, D), jnp.float32),
)

_tc_mid = pl.pallas_call(
    _tc_mid_body,
    grid=(N // BR,),
    in_specs=[_cnt_spec, _acc_spec, _row_spec, _b_spec, _w_spec],
    out_specs=_row_spec,
    out_shape=jax.ShapeDtypeStruct((N, D), jnp.float32),
)

_tc_final = pl.pallas_call(
    _tc_final_body,
    grid=(N // BR,),
    in_specs=[_cnt_spec, _acc_spec, _row_spec, _b_spec,
              pl.BlockSpec((D, 8), lambda i: (0, 0)),
              pl.BlockSpec((1, 8), lambda i: (0, 0))],
    out_specs=pl.BlockSpec((BR, 8), lambda i: (i, 0)),
    out_shape=jax.ShapeDtypeStruct((N, 8), jnp.float32),
)


def kernel(x, edge_index, W1, b1, W2, b2, Wfc, bfc):
    src2 = edge_index[0].reshape(NW, ROWS_W, CH)
    dst2 = edge_index[1].reshape(NW, ROWS_W, CH)
    wfc_p = jnp.zeros((D, 8), jnp.float32).at[:, :4].set(Wfc)
    bfc_p = jnp.zeros((1, 8), jnp.float32).at[0, :4].set(bfc)

    cnt = _sc_counts(dst2).T

    g1 = _tc_scale_matmul(cnt, x, W1)
    acc1 = _sc_scatter(g1, src2, dst2)
    g2 = _tc_mid(cnt, acc1, g1, b1.reshape(1, D), W2)
    acc2 = _sc_scatter(g2, src2, dst2)
    out = _tc_final(cnt, acc2, g2, b2.reshape(1, D), wfc_p, bfc_p)
    return out[:, :4]
